# baseline (speedup 1.0000x reference)
import jax
import jax.numpy as jnp
from jax.experimental import pallas as pl
from jax.experimental.pallas import tpu as pltpu

N = 10000
D = 128
BM = 200


def _gcn_block(scal_ref, adj_ref, inp_ref, h0_ref, w_ref, out_ref):
    a = adj_ref[...].astype(jnp.bfloat16)
    hi = jnp.dot(a, inp_ref[...], preferred_element_type=jnp.float32)
    theta = scal_ref[0]
    alpha = scal_ref[1]
    support = (1.0 - alpha) * hi + alpha * h0_ref[...]
    sw = jax.lax.dot(support, w_ref[...],
                     precision=jax.lax.Precision.HIGHEST)
    out_ref[...] = theta * sw + (1.0 - theta) * support


def kernel(input, adj, h0, weight, lamda, alpha, l):
    lam = jnp.asarray(lamda, jnp.float32)
    alp = jnp.asarray(alpha, jnp.float32)
    ell = jnp.asarray(l, jnp.float32)
    theta = jnp.log(lam / ell + 1.0)
    scal = jnp.stack([theta, alp, jnp.float32(0), jnp.float32(0)])
    inp_bf16 = input.astype(jnp.bfloat16)

    return pl.pallas_call(
        _gcn_block,
        grid=(N // BM,),
        in_specs=[
            pl.BlockSpec(memory_space=pltpu.SMEM),
            pl.BlockSpec((BM, N), lambda i: (i, 0)),
            pl.BlockSpec((N, D), lambda i: (0, 0)),
            pl.BlockSpec((BM, D), lambda i: (i, 0)),
            pl.BlockSpec((D, D), lambda i: (0, 0)),
        ],
        out_specs=pl.BlockSpec((BM, D), lambda i: (i, 0)),
        out_shape=jax.ShapeDtypeStruct((N, D), jnp.float32),
        compiler_params=pltpu.CompilerParams(
            dimension_semantics=("arbitrary",),
        ),
    )(scal, adj, inp_bf16, h0, weight)


# speedup vs baseline: 1.0572x; 1.0572x over previous
import jax
import jax.numpy as jnp
from jax.experimental import pallas as pl
from jax.experimental.pallas import tpu as pltpu

N = 10000
D = 128
BM = 400


def _gcn_block(scal_ref, adj_ref, inp_ref, h0_ref, w_ref, out_ref):
    a = adj_ref[...].astype(jnp.bfloat16)
    hi = jnp.dot(a, inp_ref[...], preferred_element_type=jnp.float32)
    theta = scal_ref[0]
    alpha = scal_ref[1]
    support = (1.0 - alpha) * hi + alpha * h0_ref[...]
    sw = jax.lax.dot(support, w_ref[...],
                     precision=jax.lax.Precision.HIGHEST)
    out_ref[...] = theta * sw + (1.0 - theta) * support


def kernel(input, adj, h0, weight, lamda, alpha, l):
    lam = jnp.asarray(lamda, jnp.float32)
    alp = jnp.asarray(alpha, jnp.float32)
    ell = jnp.asarray(l, jnp.float32)
    theta = jnp.log(lam / ell + 1.0)
    scal = jnp.stack([theta, alp, jnp.float32(0), jnp.float32(0)])
    inp_bf16 = input.astype(jnp.bfloat16)

    return pl.pallas_call(
        _gcn_block,
        grid=(N // BM,),
        in_specs=[
            pl.BlockSpec(memory_space=pltpu.SMEM),
            pl.BlockSpec((BM, N), lambda i: (i, 0)),
            pl.BlockSpec((N, D), lambda i: (0, 0)),
            pl.BlockSpec((BM, D), lambda i: (i, 0)),
            pl.BlockSpec((D, D), lambda i: (0, 0)),
        ],
        out_specs=pl.BlockSpec((BM, D), lambda i: (i, 0)),
        out_shape=jax.ShapeDtypeStruct((N, D), jnp.float32),
        compiler_params=pltpu.CompilerParams(
            dimension_semantics=("arbitrary",),
        ),
    )(scal, adj, inp_bf16, h0, weight)
